# baseline (speedup 1.0000x reference)
import functools

import jax
import jax.numpy as jnp
from jax import lax
from jax.experimental import pallas as pl
from jax.experimental.pallas import tpu as pltpu
from jax.experimental.pallas import tpu_sc as plsc

N = 10000
E = 320000
F_IN = 128
HID = 16
NCLS = 7
R = 10

NC = 2
NS = 16
NW = NC * NS
CHUNK = 128
SUPER = 4
SCHUNK = CHUNK * SUPER
SCH_PER_W = 21
CHUNKS_PER_W = SCH_PER_W * SUPER
E_PAD = NW * SCH_PER_W * SCHUNK
N_ACC = 10240
ROWS_PER_TILE = N_ACC // NS
BN = 2000
GRID_N = N // BN


def _edge_pass(table, gidx, dst, wt):
    mesh = plsc.VectorSubcoreMesh(
        core_axis_name="c", subcore_axis_name="s", num_cores=NC, num_subcores=NS
    )

    @functools.partial(
        pl.kernel,
        out_type=jax.ShapeDtypeStruct((NC, N_ACC, HID), jnp.float32),
        mesh=mesh,
        compiler_params=pltpu.CompilerParams(use_tc_tiling_on_sc=False),
        scratch_types=[
            pltpu.VMEM((CHUNKS_PER_W, CHUNK), jnp.int32),
            pltpu.VMEM((CHUNKS_PER_W, CHUNK), jnp.int32),
            pltpu.VMEM((CHUNKS_PER_W * CHUNK,), jnp.float32),
            pltpu.VMEM((SCHUNK, HID), jnp.float32),
            pltpu.VMEM((SCHUNK, HID), jnp.float32),
            pltpu.VMEM((SCHUNK, HID), jnp.float32),
            pltpu.VMEM((ROWS_PER_TILE, HID), jnp.float32),
            pltpu.VMEM_SHARED((N_ACC, HID), jnp.float32),
            pltpu.SemaphoreType.DMA,
            pltpu.SemaphoreType.DMA,
            pltpu.SemaphoreType.DMA,
            pltpu.SemaphoreType.DMA,
            pltpu.SemaphoreType.DMA,
            pltpu.SemaphoreType.DMA,
        ],
    )
    def k(table_hbm, gidx_hbm, dst_hbm, wt_hbm, out_hbm,
          gidx_v, dst_v, wt_v, rows0_v, rows1_v, rows2_v, zbuf_v, acc_sh,
          gsem0, gsem1, gsem2, ssem0, ssem1, ssem2):
        rows = (rows0_v, rows1_v, rows2_v)
        gsem = (gsem0, gsem1, gsem2)
        ssem = (ssem0, ssem1, ssem2)
        c = lax.axis_index("c")
        s = lax.axis_index("s")
        w = c * NS + s

        def zrow(i, carry):
            zbuf_v[i] = jnp.zeros((HID,), jnp.float32)
            return carry

        lax.fori_loop(0, ROWS_PER_TILE, zrow, 0)
        pltpu.sync_copy(
            zbuf_v, acc_sh.at[pl.ds(s * ROWS_PER_TILE, ROWS_PER_TILE)]
        )

        pltpu.sync_copy(gidx_hbm.at[w], gidx_v)
        pltpu.sync_copy(dst_hbm.at[w], dst_v)
        pltpu.sync_copy(wt_hbm.at[w], wt_v)
        plsc.subcore_barrier()

        def issue_gathers(slot, J):
            for q in range(SUPER):
                pltpu.async_copy(
                    table_hbm.at[gidx_v.at[J * SUPER + q]],
                    rows[slot].at[pl.ds(q * CHUNK, CHUNK)], gsem[slot])

        def wait_gathers(slot, J):
            for q in range(SUPER):
                pltpu.make_async_copy(
                    table_hbm.at[gidx_v.at[J * SUPER + q]],
                    rows[slot].at[pl.ds(q * CHUNK, CHUNK)], gsem[slot]).wait()

        def issue_scatters(slot, J):
            for q in range(SUPER):
                pltpu.async_copy(
                    rows[slot].at[pl.ds(q * CHUNK, CHUNK)],
                    acc_sh.at[dst_v.at[J * SUPER + q]], ssem[slot], add=True)

        def wait_scatters(slot, J):
            for q in range(SUPER):
                pltpu.make_async_copy(
                    rows[slot].at[pl.ds(q * CHUNK, CHUNK)],
                    acc_sh.at[dst_v.at[J * SUPER + q]], ssem[slot]).wait()

        def mul_super(J, slot):
            buf = rows[slot]

            def mul16(g, c2):
                wv = wt_v[pl.ds(J * SCHUNK + g * HID, HID)]
                for kk in range(HID):
                    buf[g * HID + kk] = buf[g * HID + kk] * wv[kk]
                return c2

            lax.fori_loop(0, SCHUNK // HID, mul16, 0)

        issue_gathers(0, 0)
        issue_gathers(1, 1)
        wait_gathers(0, 0)
        mul_super(0, 0)
        issue_scatters(0, 0)
        issue_gathers(2, 2)
        wait_gathers(1, 1)
        mul_super(1, 1)
        issue_scatters(1, 1)

        def stage(j, b):
            b_free = (b + 1) % 3
            jn = jnp.minimum(j + 1, SCH_PER_W - 1)
            wait_scatters(b_free, j)
            issue_gathers(b_free, jn)
            wait_gathers(b, j)
            mul_super(j, b)
            issue_scatters(b, j)

        stage(2, 2)

        def triple(p, carry):
            for b in (0, 1, 2):
                stage(3 * p + b, b)
            return carry

        lax.fori_loop(1, SCH_PER_W // 3, triple, 0)
        jl = SCH_PER_W - 1
        wait_gathers(0, jl)
        wait_scatters(1, jl)
        wait_scatters(2, jl)
        plsc.subcore_barrier()
        pltpu.sync_copy(
            acc_sh.at[pl.ds(s * ROWS_PER_TILE, ROWS_PER_TILE)],
            out_hbm.at[c, pl.ds(s * ROWS_PER_TILE, ROWS_PER_TILE)],
        )

    return k(table, gidx, dst, wt)


def _tc_transform1(x, wrel, wroot):
    def body(x_ref, w_ref, r_ref, rel_ref, root_ref):
        xb = x_ref[...]
        rel_ref[...] = jnp.dot(xb, w_ref[...], preferred_element_type=jnp.float32)
        root_ref[...] = jnp.dot(xb, r_ref[...], preferred_element_type=jnp.float32)

    return pl.pallas_call(
        body,
        grid=(GRID_N,),
        in_specs=[
            pl.BlockSpec((BN, F_IN), lambda i: (i, 0)),
            pl.BlockSpec((F_IN, R * HID), lambda i: (0, 0)),
            pl.BlockSpec((F_IN, HID), lambda i: (0, 0)),
        ],
        out_specs=[
            pl.BlockSpec((BN, R * HID), lambda i: (i, 0)),
            pl.BlockSpec((BN, HID), lambda i: (i, 0)),
        ],
        out_shape=[
            jax.ShapeDtypeStruct((N, R * HID), jnp.float32),
            jax.ShapeDtypeStruct((N, HID), jnp.float32),
        ],
    )(x, wrel, wroot)


def _tc_mid(agg1, troot, b1, wrel2, root2p):
    def body(agg_ref, troot_ref, b1_ref, w_ref, r_ref, rel_ref, hroot_ref):
        h = agg_ref[0] + agg_ref[1] + troot_ref[...] + b1_ref[...]
        h = jnp.maximum(h, 0.0)
        rel_ref[...] = jnp.dot(h, w_ref[...], preferred_element_type=jnp.float32)
        hroot_ref[...] = jnp.dot(h, r_ref[...], preferred_element_type=jnp.float32)

    return pl.pallas_call(
        body,
        grid=(GRID_N,),
        in_specs=[
            pl.BlockSpec((NC, BN, HID), lambda i: (0, i, 0)),
            pl.BlockSpec((BN, HID), lambda i: (i, 0)),
            pl.BlockSpec((1, HID), lambda i: (0, 0)),
            pl.BlockSpec((HID, R * HID), lambda i: (0, 0)),
            pl.BlockSpec((HID, HID), lambda i: (0, 0)),
        ],
        out_specs=[
            pl.BlockSpec((BN, R * HID), lambda i: (i, 0)),
            pl.BlockSpec((BN, HID), lambda i: (i, 0)),
        ],
        out_shape=[
            jax.ShapeDtypeStruct((N, R * HID), jnp.float32),
            jax.ShapeDtypeStruct((N, HID), jnp.float32),
        ],
    )(agg1, troot, b1, wrel2, root2p)


def _tc_final(agg2, hroot, b2p):
    def body(agg_ref, hroot_ref, b2_ref, ls_ref, o_ref):
        o = agg_ref[0] + agg_ref[1] + hroot_ref[...] + b2_ref[...]
        col = lax.broadcasted_iota(jnp.int32, (BN, HID), 1)
        mask = col < NCLS
        om = jnp.where(mask, o, jnp.float32(-1e30))
        m = jnp.max(om, axis=1, keepdims=True)
        e = jnp.where(mask, jnp.exp(o - m), 0.0)
        ssum = jnp.sum(e, axis=1, keepdims=True)
        ls_ref[...] = o - m - jnp.log(ssum)
        o_ref[...] = o

    return pl.pallas_call(
        body,
        grid=(GRID_N,),
        in_specs=[
            pl.BlockSpec((NC, BN, HID), lambda i: (0, i, 0)),
            pl.BlockSpec((BN, HID), lambda i: (i, 0)),
            pl.BlockSpec((1, HID), lambda i: (0, 0)),
        ],
        out_specs=[
            pl.BlockSpec((BN, HID), lambda i: (i, 0)),
            pl.BlockSpec((BN, HID), lambda i: (i, 0)),
        ],
        out_shape=[
            jax.ShapeDtypeStruct((N, HID), jnp.float32),
            jax.ShapeDtypeStruct((N, HID), jnp.float32),
        ],
    )(agg2, hroot, b2p)


def kernel(x, edge_index, edge_weight, edge_color, W1, root1, b1, W2, root2, b2):
    src = edge_index[0].astype(jnp.int32)
    dst = edge_index[1].astype(jnp.int32)
    col = edge_color.astype(jnp.int32)
    gidx = src * R + col
    pad = E_PAD - E
    gidx = jnp.pad(gidx, (0, pad)).reshape(NW, CHUNKS_PER_W, CHUNK)
    dstp = jnp.pad(dst, (0, pad)).reshape(NW, CHUNKS_PER_W, CHUNK)
    wtp = jnp.pad(edge_weight, (0, pad)).reshape(NW, CHUNKS_PER_W * CHUNK)

    wrel1 = W1.transpose(1, 0, 2).reshape(F_IN, R * HID)
    t_rel1, t_root1 = _tc_transform1(x, wrel1, root1)
    agg1 = _edge_pass(t_rel1.reshape(N * R, HID), gidx, dstp, wtp)

    wrel2 = jnp.pad(W2, ((0, 0), (0, 0), (0, HID - NCLS)))
    wrel2 = wrel2.transpose(1, 0, 2).reshape(HID, R * HID)
    root2p = jnp.pad(root2, ((0, 0), (0, HID - NCLS)))
    t_rel2, t_hroot = _tc_mid(agg1, t_root1, b1.reshape(1, HID), wrel2, root2p)
    agg2 = _edge_pass(t_rel2.reshape(N * R, HID), gidx, dstp, wtp)

    b2p = jnp.pad(b2, (0, HID - NCLS)).reshape(1, HID)
    ls, o = _tc_final(agg2, t_hroot, b2p)
    return (ls[:, :NCLS], o[:, :NCLS])


# speedup vs baseline: 1.3781x; 1.3781x over previous
import functools

import jax
import jax.numpy as jnp
from jax import lax
from jax.experimental import pallas as pl
from jax.experimental.pallas import tpu as pltpu
from jax.experimental.pallas import tpu_sc as plsc

N = 10000
E = 320000
F_IN = 128
HID = 16
NCLS = 7
R = 10

NC = 2
NS = 16
NW = NC * NS
CHUNK = 128
SUPER = 2
SCHUNK = CHUNK * SUPER
SCH_A = 51
SCH_B = 30
CPW = SCH_A * SUPER
EA = SCH_A * SCHUNK
EB = SCH_B * SCHUNK
E_PAD = NS * (EA + EB)
N_ACC = 10240
ROWS_PER_TILE = N_ACC // NS
BN = 2000
GRID_N = N // BN


def _edge_pass(table, gidx, dst, wt):
    mesh = plsc.VectorSubcoreMesh(
        core_axis_name="c", subcore_axis_name="s", num_cores=NC, num_subcores=NS
    )

    @functools.partial(
        pl.kernel,
        out_type=jax.ShapeDtypeStruct((NC, N_ACC, HID), jnp.float32),
        mesh=mesh,
        compiler_params=pltpu.CompilerParams(use_tc_tiling_on_sc=False),
        scratch_types=[
            pltpu.VMEM((CPW, CHUNK), jnp.int32),
            pltpu.VMEM((CPW, CHUNK), jnp.int32),
            pltpu.VMEM((CPW * CHUNK,), jnp.float32),
            pltpu.VMEM((SCHUNK, HID), jnp.float32),
            pltpu.VMEM((SCHUNK, HID), jnp.float32),
            pltpu.VMEM((SCHUNK, HID), jnp.float32),
            pltpu.VMEM((ROWS_PER_TILE, HID), jnp.float32),
            pltpu.VMEM_SHARED((N_ACC, HID), jnp.float32),
            pltpu.SemaphoreType.DMA,
            pltpu.SemaphoreType.DMA,
            pltpu.SemaphoreType.DMA,
            pltpu.SemaphoreType.DMA,
            pltpu.SemaphoreType.DMA,
            pltpu.SemaphoreType.DMA,
        ],
    )
    def k(table_hbm, gidx_hbm, dst_hbm, wt_hbm, out_hbm,
          gidx_v, dst_v, wt_v, rows0_v, rows1_v, rows2_v, zbuf_v, acc_sh,
          gsem0, gsem1, gsem2, ssem0, ssem1, ssem2):
        rows = (rows0_v, rows1_v, rows2_v)
        gsem = (gsem0, gsem1, gsem2)
        ssem = (ssem0, ssem1, ssem2)
        c = lax.axis_index("c")
        s = lax.axis_index("s")
        w = c * NS + s

        def zrow(i, carry):
            zbuf_v[i] = jnp.zeros((HID,), jnp.float32)
            return carry

        lax.fori_loop(0, ROWS_PER_TILE, zrow, 0)
        pltpu.sync_copy(
            zbuf_v, acc_sh.at[pl.ds(s * ROWS_PER_TILE, ROWS_PER_TILE)]
        )
        plsc.subcore_barrier()

        def issue_gathers(slot, J):
            for q in range(SUPER):
                pltpu.async_copy(
                    table_hbm.at[gidx_v.at[J * SUPER + q]],
                    rows[slot].at[pl.ds(q * CHUNK, CHUNK)], gsem[slot])

        def wait_gathers(slot, J):
            for q in range(SUPER):
                pltpu.make_async_copy(
                    table_hbm.at[gidx_v.at[J * SUPER + q]],
                    rows[slot].at[pl.ds(q * CHUNK, CHUNK)], gsem[slot]).wait()

        def issue_scatters(slot, J):
            for q in range(SUPER):
                pltpu.async_copy(
                    rows[slot].at[pl.ds(q * CHUNK, CHUNK)],
                    acc_sh.at[dst_v.at[J * SUPER + q]], ssem[slot], add=True)

        def wait_scatters(slot, J):
            for q in range(SUPER):
                pltpu.make_async_copy(
                    rows[slot].at[pl.ds(q * CHUNK, CHUNK)],
                    acc_sh.at[dst_v.at[J * SUPER + q]], ssem[slot]).wait()

        def mul_super(J, slot):
            buf = rows[slot]

            def mul16(g, c2):
                wv = wt_v[pl.ds(J * SCHUNK + g * HID, HID)]
                for kk in range(HID):
                    buf[g * HID + kk] = buf[g * HID + kk] * wv[kk]
                return c2

            lax.fori_loop(0, SCHUNK // HID, mul16, 0)

        def pipeline(nstages):
            nchunks = nstages * SUPER
            pltpu.sync_copy(gidx_hbm.at[w, pl.ds(0, nchunks)],
                            gidx_v.at[pl.ds(0, nchunks)])
            pltpu.sync_copy(dst_hbm.at[w, pl.ds(0, nchunks)],
                            dst_v.at[pl.ds(0, nchunks)])
            pltpu.sync_copy(wt_hbm.at[w, pl.ds(0, nchunks * CHUNK)],
                            wt_v.at[pl.ds(0, nchunks * CHUNK)])

            def stage(j, b):
                b_free = (b + 1) % 3
                jn = jnp.minimum(j + 1, nstages - 1)
                wait_scatters(b_free, j)
                issue_gathers(b_free, jn)
                wait_gathers(b, j)
                mul_super(j, b)
                issue_scatters(b, j)

            issue_gathers(0, 0)
            issue_gathers(1, jnp.minimum(1, nstages - 1))
            wait_gathers(0, 0)
            mul_super(0, 0)
            issue_scatters(0, 0)
            issue_gathers(2, jnp.minimum(2, nstages - 1))
            wait_gathers(1, 1)
            mul_super(1, 1)
            issue_scatters(1, 1)
            stage(2, 2)

            def triple(p, carry):
                for b in (0, 1, 2):
                    stage(3 * p + b, b)
                return carry

            lax.fori_loop(1, nstages // 3, triple, 0)
            jl = nstages - 1
            wait_gathers(0, jl)
            wait_scatters(1, jl)
            wait_scatters(2, jl)

        @pl.when(c == 0)
        def _():
            pipeline(SCH_A)

        @pl.when(c != 0)
        def _():
            pipeline(SCH_B)

        plsc.subcore_barrier()
        pltpu.sync_copy(
            acc_sh.at[pl.ds(s * ROWS_PER_TILE, ROWS_PER_TILE)],
            out_hbm.at[c, pl.ds(s * ROWS_PER_TILE, ROWS_PER_TILE)],
        )

    return k(table, gidx, dst, wt)


def _tc_transform1(x, wrel, wroot):
    def body(x_ref, w_ref, r_ref, rel_ref, root_ref):
        xb = x_ref[...]
        rel_ref[...] = jnp.dot(xb, w_ref[...], preferred_element_type=jnp.float32)
        root_ref[...] = jnp.dot(xb, r_ref[...], preferred_element_type=jnp.float32)

    return pl.pallas_call(
        body,
        grid=(GRID_N,),
        in_specs=[
            pl.BlockSpec((BN, F_IN), lambda i: (i, 0)),
            pl.BlockSpec((F_IN, R * HID), lambda i: (0, 0)),
            pl.BlockSpec((F_IN, HID), lambda i: (0, 0)),
        ],
        out_specs=[
            pl.BlockSpec((BN, R * HID), lambda i: (i, 0)),
            pl.BlockSpec((BN, HID), lambda i: (i, 0)),
        ],
        out_shape=[
            jax.ShapeDtypeStruct((N, R * HID), jnp.float32),
            jax.ShapeDtypeStruct((N, HID), jnp.float32),
        ],
    )(x, wrel, wroot)


def _tc_mid(agg1, troot, b1, wrel2, root2p):
    def body(agg_ref, troot_ref, b1_ref, w_ref, r_ref, rel_ref, hroot_ref):
        h = agg_ref[0] + agg_ref[1] + troot_ref[...] + b1_ref[...]
        h = jnp.maximum(h, 0.0)
        rel_ref[...] = jnp.dot(h, w_ref[...], preferred_element_type=jnp.float32)
        hroot_ref[...] = jnp.dot(h, r_ref[...], preferred_element_type=jnp.float32)

    return pl.pallas_call(
        body,
        grid=(GRID_N,),
        in_specs=[
            pl.BlockSpec((NC, BN, HID), lambda i: (0, i, 0)),
            pl.BlockSpec((BN, HID), lambda i: (i, 0)),
            pl.BlockSpec((1, HID), lambda i: (0, 0)),
            pl.BlockSpec((HID, R * HID), lambda i: (0, 0)),
            pl.BlockSpec((HID, HID), lambda i: (0, 0)),
        ],
        out_specs=[
            pl.BlockSpec((BN, R * HID), lambda i: (i, 0)),
            pl.BlockSpec((BN, HID), lambda i: (i, 0)),
        ],
        out_shape=[
            jax.ShapeDtypeStruct((N, R * HID), jnp.float32),
            jax.ShapeDtypeStruct((N, HID), jnp.float32),
        ],
    )(agg1, troot, b1, wrel2, root2p)


def _tc_final(agg2, hroot, b2p):
    def body(agg_ref, hroot_ref, b2_ref, ls_ref, o_ref):
        o = agg_ref[0] + agg_ref[1] + hroot_ref[...] + b2_ref[...]
        col = lax.broadcasted_iota(jnp.int32, (BN, HID), 1)
        mask = col < NCLS
        om = jnp.where(mask, o, jnp.float32(-1e30))
        m = jnp.max(om, axis=1, keepdims=True)
        e = jnp.where(mask, jnp.exp(o - m), 0.0)
        ssum = jnp.sum(e, axis=1, keepdims=True)
        ls_ref[...] = o - m - jnp.log(ssum)
        o_ref[...] = o

    return pl.pallas_call(
        body,
        grid=(GRID_N,),
        in_specs=[
            pl.BlockSpec((NC, BN, HID), lambda i: (0, i, 0)),
            pl.BlockSpec((BN, HID), lambda i: (i, 0)),
            pl.BlockSpec((1, HID), lambda i: (0, 0)),
        ],
        out_specs=[
            pl.BlockSpec((BN, HID), lambda i: (i, 0)),
            pl.BlockSpec((BN, HID), lambda i: (i, 0)),
        ],
        out_shape=[
            jax.ShapeDtypeStruct((N, HID), jnp.float32),
            jax.ShapeDtypeStruct((N, HID), jnp.float32),
        ],
    )(agg2, hroot, b2p)


def kernel(x, edge_index, edge_weight, edge_color, W1, root1, b1, W2, root2, b2):
    src = edge_index[0].astype(jnp.int32)
    dst = edge_index[1].astype(jnp.int32)
    col = edge_color.astype(jnp.int32)
    gidx = src * R + col

    def dist(a):
        a = jnp.pad(a, (0, E_PAD - E))
        pa = a[: NS * EA].reshape(NS, EA)
        pb = jnp.pad(a[NS * EA:].reshape(NS, EB), ((0, 0), (0, EA - EB)))
        return jnp.concatenate([pa, pb], axis=0)

    gidx = dist(gidx).reshape(NW, CPW, CHUNK)
    dstp = dist(dst).reshape(NW, CPW, CHUNK)
    wtp = dist(edge_weight)

    wrel1 = W1.transpose(1, 0, 2).reshape(F_IN, R * HID)
    t_rel1, t_root1 = _tc_transform1(x, wrel1, root1)
    agg1 = _edge_pass(t_rel1.reshape(N * R, HID), gidx, dstp, wtp)

    wrel2 = jnp.pad(W2, ((0, 0), (0, 0), (0, HID - NCLS)))
    wrel2 = wrel2.transpose(1, 0, 2).reshape(HID, R * HID)
    root2p = jnp.pad(root2, ((0, 0), (0, HID - NCLS)))
    t_rel2, t_hroot = _tc_mid(agg1, t_root1, b1.reshape(1, HID), wrel2, root2p)
    agg2 = _edge_pass(t_rel2.reshape(N * R, HID), gidx, dstp, wtp)

    b2p = jnp.pad(b2, (0, HID - NCLS)).reshape(1, HID)
    ls, o = _tc_final(agg2, t_hroot, b2p)
    return (ls[:, :NCLS], o[:, :NCLS])
